# baseline (speedup 1.0000x reference)
import functools

import jax
import jax.numpy as jnp
from jax import lax
from jax.experimental import pallas as pl
from jax.experimental.pallas import tpu as pltpu
from jax.experimental.pallas import tpu_sc as plsc

N = 10000
D_IN = 128
H = 32
K = 10
C_OUT = 7
E = 160000

NC = 2
NS = 16
NW = NC * NS
CH = 128
EWP = 5120
EP = EWP * NW
NCHUNK = EWP // CH
KG = 8
NG = NCHUNK // KG
TOTCH = EP // CH
NCH0 = 40
NCH1 = 80 - NCH0
C0TOT = NS * NCH0
NCHMAX = max(NCH0, NCH1)
NP = 10240
RPS = NP // NS
BN = 2048
GRID = NP // BN


@functools.cache
def _sc_mesh():
    return plsc.VectorSubcoreMesh(core_axis_name="c", subcore_axis_name="s",
                                  num_cores=NC, num_subcores=NS)


@functools.cache
def _sc_degrees_kernel():
    @functools.partial(
        pl.kernel,
        out_type=jax.ShapeDtypeStruct((NC, NP, 16), jnp.float32),
        mesh=_sc_mesh(),
        compiler_params=pltpu.CompilerParams(use_tc_tiling_on_sc=False),
        scratch_types=[
            pltpu.VMEM((NCHUNK, CH), jnp.int32),
            pltpu.VMEM((CH, 16), jnp.float32),
            pltpu.VMEM_SHARED((NP, 16), jnp.float32),
            pltpu.SemaphoreType.DMA,
        ],
    )
    def _sc_degrees(dst_h, ones_h, zeros_h, degin_h,
                    di_v, ones_v, accin_s, sem_in):
        c = lax.axis_index("c")
        s = lax.axis_index("s")
        wid = s * NC + c
        pltpu.sync_copy(zeros_h.at[pl.ds(s * RPS, RPS)], accin_s.at[pl.ds(s * RPS, RPS)])
        pltpu.sync_copy(ones_h, ones_v)
        pltpu.sync_copy(dst_h.at[pl.ds(wid * NCHUNK, NCHUNK)], di_v)
        plsc.subcore_barrier()

        for g in range(NG):
            ds = []
            for j in range(KG):
                t = g * KG + j
                ds.append(pltpu.async_copy(ones_v, accin_s.at[di_v.at[t]],
                                           sem_in, add=True))
            for d in ds:
                d.wait()
        plsc.subcore_barrier()
        pltpu.sync_copy(accin_s.at[pl.ds(s * RPS, RPS)], degin_h.at[c, pl.ds(s * RPS, RPS)])

    return _sc_degrees


@functools.cache
def _segsum_kernel(width):
    @functools.partial(
        pl.kernel,
        out_type=jax.ShapeDtypeStruct((NC, NP, width), jnp.float32),
        mesh=_sc_mesh(),
        compiler_params=pltpu.CompilerParams(use_tc_tiling_on_sc=False),
        scratch_types=[
            pltpu.VMEM((NCHUNK, CH), jnp.int32),
            pltpu.VMEM((NCHUNK, CH), jnp.int32),
            pltpu.VMEM((2 * KG, CH, width), jnp.float32),
            pltpu.VMEM((CH, width), jnp.float32),
            pltpu.VMEM_SHARED((NP, width), jnp.float32),
            pltpu.SemaphoreType.DMA,
            pltpu.SemaphoreType.DMA,
        ],
    )
    def _segsum(tab_h, gidx_h, sidx_h, zeros_h, out_h,
                gi_v, si_v, rows_v, zrows_v, acc_s, gsem, ssem):
        c = lax.axis_index("c")
        s = lax.axis_index("s")
        wid = s * NC + c
        base = wid * NCHUNK
        pltpu.sync_copy(zeros_h.at[pl.ds(s * RPS, RPS)], acc_s.at[pl.ds(s * RPS, RPS)])
        pltpu.sync_copy(gidx_h.at[pl.ds(base, NCHUNK)], gi_v)
        pltpu.sync_copy(sidx_h.at[pl.ds(base, NCHUNK)], si_v)
        pltpu.sync_copy(zeros_h.at[pl.ds(0, CH)], zrows_v)
        plsc.subcore_barrier()

        def gwait():
            pltpu.make_async_copy(tab_h.at[gi_v.at[0]], rows_v.at[0], gsem).wait()

        def swait():
            pltpu.make_async_copy(zeros_h.at[pl.ds(0, CH)], zrows_v, ssem).wait()

        for j in range(KG):
            pltpu.async_copy(tab_h.at[gi_v.at[j]], rows_v.at[j], gsem)
            pltpu.async_copy(zrows_v, acc_s.at[si_v.at[0]], ssem, add=True)

        def body(g, carry):
            h = (g % 2) * KG
            nh = ((g + 1) % 2) * KG
            for j in range(KG):
                swait()
            for j in range(KG):
                t = jnp.minimum((g + 1) * KG + j, NCHUNK - 1)
                pltpu.async_copy(tab_h.at[gi_v.at[t]], rows_v.at[nh + j], gsem)
            for j in range(KG):
                gwait()
            for j in range(KG):
                pltpu.async_copy(rows_v.at[h + j], acc_s.at[si_v.at[g * KG + j]],
                                 ssem, add=True)
            return carry

        lax.fori_loop(0, NG, body, 0)
        for j in range(KG):
            gwait()
            swait()

        plsc.subcore_barrier()
        pltpu.sync_copy(acc_s.at[pl.ds(s * RPS, RPS)], out_h.at[c, pl.ds(s * RPS, RPS)])

    return _segsum


def _t1_body(degin_ref, x_ref, w1_ref, g1_ref, dis_ref):
    deg = degin_ref[0] + degin_ref[1] + 1.0
    dis = lax.rsqrt(deg)
    dis_ref[...] = dis
    h = jnp.dot(x_ref[...], w1_ref[...], preferred_element_type=jnp.float32)
    g1_ref[...] = dis[:, 0:1] * h


def _t2_body(seg_ref, g1_ref, dis_ref, b1_ref, w2_ref, g2_ref):
    dis = dis_ref[...][:, 0:1]
    h1 = jnp.maximum(dis * (seg_ref[0] + seg_ref[1] + g1_ref[...]) + b1_ref[...], 0.0)
    g2_ref[...] = dis * jnp.dot(h1, w2_ref[...], preferred_element_type=jnp.float32)


def _t3_body(seg_ref, g2_ref, dis_ref, b2_ref, wp_ref, bp_ref,
             wc_ref, bc_ref, y_ref, spad_ref, ss_ref):
    i = pl.program_id(0)
    dis = dis_ref[...][:, 0:1]
    h2 = jnp.maximum(dis * (seg_ref[0] + seg_ref[1] + g2_ref[...]) + b2_ref[...], 0.0)
    sl = jnp.dot(h2, wp_ref[...], preferred_element_type=jnp.float32) + bp_ref[...]
    sm = jax.nn.softmax(sl, axis=-1)
    rows = i * BN + lax.broadcasted_iota(jnp.int32, (BN, 1), 0)
    sm = jnp.where(rows < N, sm, 0.0)
    spad_ref[...] = jnp.concatenate(
        [sm, jnp.zeros((BN, 15 - K), jnp.float32), jnp.ones((BN, 1), jnp.float32)],
        axis=1)
    logits = jnp.dot(h2, wc_ref[...], preferred_element_type=jnp.float32) + bc_ref[...]
    y_ref[...] = jax.nn.log_softmax(logits, axis=-1)

    @pl.when(i == 0)
    def _init():
        ss_ref[...] = jnp.zeros_like(ss_ref)

    ss_ref[...] += lax.dot_general(sm, sm, (((0,), (0,)), ((), ())),
                                   preferred_element_type=jnp.float32)


def _t4_body(spad_ref, as_ref, ss_ref, mc_ref, ol_ref, num_acc, den_acc):
    i = pl.program_id(0)

    @pl.when(i == 0)
    def _init():
        num_acc[0, 0] = 0.0
        den_acc[0, 0] = 0.0

    sv = spad_ref[...][:, :K]
    av = as_ref[0] + as_ref[1]
    num_acc[0, 0] += jnp.sum(sv * av[:, :K])
    den_acc[0, 0] += jnp.sum(av[:, 15:16] * jnp.sum(sv * sv, axis=1, keepdims=True))

    @pl.when(i == pl.num_programs(0) - 1)
    def _fin():
        mc_ref[0, 0] = -(num_acc[0, 0] / den_acc[0, 0])
        ssv = ss_ref[...]
        nss = jnp.sqrt(jnp.sum(ssv * ssv))
        r = lax.broadcasted_iota(jnp.int32, (K, K), 0)
        cc = lax.broadcasted_iota(jnp.int32, (K, K), 1)
        eye = jnp.where(r == cc, 1.0 / jnp.sqrt(jnp.float32(K)), 0.0)
        t = ssv / nss - eye
        ol_ref[0, 0] = jnp.sqrt(jnp.sum(t * t))


_t1 = pl.pallas_call(
    _t1_body,
    grid=(GRID,),
    in_specs=[
        pl.BlockSpec((NC, BN, 16), lambda i: (0, i, 0)),
        pl.BlockSpec((BN, D_IN), lambda i: (i, 0)),
        pl.BlockSpec((D_IN, H), lambda i: (0, 0)),
    ],
    out_specs=[
        pl.BlockSpec((BN, H), lambda i: (i, 0)),
        pl.BlockSpec((BN, 16), lambda i: (i, 0)),
    ],
    out_shape=[
        jax.ShapeDtypeStruct((NP, H), jnp.float32),
        jax.ShapeDtypeStruct((NP, 16), jnp.float32),
    ],
)

_t2 = pl.pallas_call(
    _t2_body,
    grid=(GRID,),
    in_specs=[
        pl.BlockSpec((NC, BN, H), lambda i: (0, i, 0)),
        pl.BlockSpec((BN, H), lambda i: (i, 0)),
        pl.BlockSpec((BN, 16), lambda i: (i, 0)),
        pl.BlockSpec((1, H), lambda i: (0, 0)),
        pl.BlockSpec((H, H), lambda i: (0, 0)),
    ],
    out_specs=pl.BlockSpec((BN, H), lambda i: (i, 0)),
    out_shape=jax.ShapeDtypeStruct((NP, H), jnp.float32),
)

_t3 = pl.pallas_call(
    _t3_body,
    grid=(GRID,),
    in_specs=[
        pl.BlockSpec((NC, BN, H), lambda i: (0, i, 0)),
        pl.BlockSpec((BN, H), lambda i: (i, 0)),
        pl.BlockSpec((BN, 16), lambda i: (i, 0)),
        pl.BlockSpec((1, H), lambda i: (0, 0)),
        pl.BlockSpec((H, K), lambda i: (0, 0)),
        pl.BlockSpec((1, K), lambda i: (0, 0)),
        pl.BlockSpec((H, C_OUT), lambda i: (0, 0)),
        pl.BlockSpec((1, C_OUT), lambda i: (0, 0)),
    ],
    out_specs=[
        pl.BlockSpec((BN, C_OUT), lambda i: (i, 0)),
        pl.BlockSpec((BN, 16), lambda i: (i, 0)),
        pl.BlockSpec((K, K), lambda i: (0, 0)),
    ],
    out_shape=[
        jax.ShapeDtypeStruct((NP, C_OUT), jnp.float32),
        jax.ShapeDtypeStruct((NP, 16), jnp.float32),
        jax.ShapeDtypeStruct((K, K), jnp.float32),
    ],
)

_t4 = pl.pallas_call(
    _t4_body,
    grid=(GRID,),
    in_specs=[
        pl.BlockSpec((BN, 16), lambda i: (i, 0)),
        pl.BlockSpec((NC, BN, 16), lambda i: (0, i, 0)),
        pl.BlockSpec((K, K), lambda i: (0, 0)),
    ],
    out_specs=[
        pl.BlockSpec(memory_space=pltpu.SMEM),
        pl.BlockSpec(memory_space=pltpu.SMEM),
    ],
    out_shape=[
        jax.ShapeDtypeStruct((1, 1), jnp.float32),
        jax.ShapeDtypeStruct((1, 1), jnp.float32),
    ],
    scratch_shapes=[pltpu.SMEM((1, 1), jnp.float32),
                    pltpu.SMEM((1, 1), jnp.float32)],
)


def kernel(x, edge_index, W1, b1, W2, b2, Wp, bp, Wc, bc):
    src = edge_index[0]
    dst = edge_index[1]
    pad = jnp.full((EP - E,), N, jnp.int32)
    srcp = jnp.concatenate([src, pad]).reshape(TOTCH, CH)
    dstp = jnp.concatenate([dst, pad]).reshape(TOTCH, CH)
    xp = jnp.pad(x, ((0, NP - N), (0, 0)))
    zeros16 = jnp.zeros((NP, 16), jnp.float32)
    zeros32 = jnp.zeros((NP, H), jnp.float32)
    ones = jnp.ones((CH, 16), jnp.float32)

    degin = _sc_degrees_kernel()(dstp, ones, zeros16)
    g1, dis = _t1(degin, xp, W1)
    seg1 = _segsum_kernel(H)(g1, srcp, dstp, zeros32)
    g2 = _t2(seg1, g1, dis, b1.reshape(1, H), W2)
    seg2 = _segsum_kernel(H)(g2, srcp, dstp, zeros32)
    y, spad, ss = _t3(seg2, g2, dis, b2.reshape(1, H),
                      Wp, bp.reshape(1, K), Wc, bc.reshape(1, C_OUT))
    a_s = _segsum_kernel(16)(spad, dstp, srcp, zeros16)
    mc, ol = _t4(spad, a_s, ss)
    return (y[:N], mc[0, 0], ol[0, 0])


# speedup vs baseline: 1.0887x; 1.0887x over previous
import functools

import jax
import jax.numpy as jnp
from jax import lax
from jax.experimental import pallas as pl
from jax.experimental.pallas import tpu as pltpu
from jax.experimental.pallas import tpu_sc as plsc

N = 10000
D_IN = 128
H = 32
K = 10
C_OUT = 7
E = 160000

NC = 2
NS = 16
NW = NC * NS
CH = 128
EWP = 5120
EP = EWP * NW
NCHUNK = EWP // CH
KG = 8
NG = NCHUNK // KG
TOTCH = EP // CH
NCH0 = 40
NCH1 = 80 - NCH0
C0TOT = NS * NCH0
NCHMAX = max(NCH0, NCH1)
NP = 10240
RPS = NP // NS
BN = 2048
GRID = NP // BN


@functools.cache
def _sc_mesh():
    return plsc.VectorSubcoreMesh(core_axis_name="c", subcore_axis_name="s",
                                  num_cores=NC, num_subcores=NS)


@functools.cache
def _sc_degrees_kernel():
    @functools.partial(
        pl.kernel,
        out_type=jax.ShapeDtypeStruct((NC, NP, 16), jnp.float32),
        mesh=_sc_mesh(),
        compiler_params=pltpu.CompilerParams(use_tc_tiling_on_sc=False),
        scratch_types=[
            pltpu.VMEM((NCHUNK, CH), jnp.int32),
            pltpu.VMEM((CH, 16), jnp.float32),
            pltpu.VMEM_SHARED((NP, 16), jnp.float32),
            pltpu.SemaphoreType.DMA,
        ],
    )
    def _sc_degrees(dst_h, ones_h, zeros_h, degin_h,
                    di_v, ones_v, accin_s, sem_in):
        c = lax.axis_index("c")
        s = lax.axis_index("s")
        wid = s * NC + c
        pltpu.sync_copy(zeros_h.at[pl.ds(s * RPS, RPS)], accin_s.at[pl.ds(s * RPS, RPS)])
        pltpu.sync_copy(ones_h, ones_v)
        pltpu.sync_copy(dst_h.at[pl.ds(wid * NCHUNK, NCHUNK)], di_v)
        plsc.subcore_barrier()

        for g in range(NG):
            ds = []
            for j in range(KG):
                t = g * KG + j
                ds.append(pltpu.async_copy(ones_v, accin_s.at[di_v.at[t]],
                                           sem_in, add=True))
            for d in ds:
                d.wait()
        plsc.subcore_barrier()
        pltpu.sync_copy(accin_s.at[pl.ds(s * RPS, RPS)], degin_h.at[c, pl.ds(s * RPS, RPS)])

    return _sc_degrees


@functools.cache
def _segsum_kernel(width):
    @functools.partial(
        pl.kernel,
        out_type=jax.ShapeDtypeStruct((NC, NP, width), jnp.float32),
        mesh=_sc_mesh(),
        compiler_params=pltpu.CompilerParams(use_tc_tiling_on_sc=False),
        scratch_types=[
            pltpu.VMEM((NCHUNK, CH), jnp.int32),
            pltpu.VMEM((NCHUNK, CH), jnp.int32),
            pltpu.VMEM((KG, CH, width), jnp.float32),
            pltpu.VMEM((KG, CH, width), jnp.float32),
            pltpu.VMEM_SHARED((NP, width), jnp.float32),
            pltpu.SemaphoreType.DMA,
            pltpu.SemaphoreType.DMA,
            pltpu.SemaphoreType.DMA,
            pltpu.SemaphoreType.DMA,
        ],
    )
    def _segsum(tab_h, gidx_h, sidx_h, zeros_h, out_h,
                gi_v, si_v, rows_a, rows_b, acc_s, gsa, gsb, ssa, ssb):
        c = lax.axis_index("c")
        s = lax.axis_index("s")
        wid = s * NC + c
        base = wid * NCHUNK
        pltpu.sync_copy(zeros_h.at[pl.ds(s * RPS, RPS)], acc_s.at[pl.ds(s * RPS, RPS)])
        pltpu.sync_copy(gidx_h.at[pl.ds(base, NCHUNK)], gi_v)
        pltpu.sync_copy(sidx_h.at[pl.ds(base, NCHUNK)], si_v)
        plsc.subcore_barrier()

        rows = (rows_a, rows_b)
        gsem = (gsa, gsb)
        ssem = (ssa, ssb)
        gdesc = [[], []]
        sdesc = [[], []]

        def fire_gathers(g):
            b = g % 2
            gdesc[b] = [
                pltpu.async_copy(tab_h.at[gi_v.at[g * KG + j]], rows[b].at[j], gsem[b])
                for j in range(KG)
            ]

        fire_gathers(0)
        for g in range(NG):
            b = g % 2
            if g + 1 < NG:
                for d in sdesc[1 - b]:
                    d.wait()
                fire_gathers(g + 1)
            for d in gdesc[b]:
                d.wait()
            sdesc[b] = [
                pltpu.async_copy(rows[b].at[j], acc_s.at[si_v.at[g * KG + j]],
                                 ssem[b], add=True)
                for j in range(KG)
            ]
        for b in (0, 1):
            for d in sdesc[b]:
                d.wait()
        plsc.subcore_barrier()
        pltpu.sync_copy(acc_s.at[pl.ds(s * RPS, RPS)], out_h.at[c, pl.ds(s * RPS, RPS)])

    return _segsum


def _t1_body(degin_ref, x_ref, w1_ref, g1_ref, dis_ref):
    deg = degin_ref[0] + degin_ref[1] + 1.0
    dis = lax.rsqrt(deg)
    dis_ref[...] = dis
    h = jnp.dot(x_ref[...], w1_ref[...], preferred_element_type=jnp.float32)
    g1_ref[...] = dis[:, 0:1] * h


def _t2_body(seg_ref, g1_ref, dis_ref, b1_ref, w2_ref, g2_ref):
    dis = dis_ref[...][:, 0:1]
    h1 = jnp.maximum(dis * (seg_ref[0] + seg_ref[1] + g1_ref[...]) + b1_ref[...], 0.0)
    g2_ref[...] = dis * jnp.dot(h1, w2_ref[...], preferred_element_type=jnp.float32)


def _t3_body(seg_ref, g2_ref, dis_ref, b2_ref, wp_ref, bp_ref,
             wc_ref, bc_ref, y_ref, spad_ref, ss_ref):
    i = pl.program_id(0)
    dis = dis_ref[...][:, 0:1]
    h2 = jnp.maximum(dis * (seg_ref[0] + seg_ref[1] + g2_ref[...]) + b2_ref[...], 0.0)
    sl = jnp.dot(h2, wp_ref[...], preferred_element_type=jnp.float32) + bp_ref[...]
    sm = jax.nn.softmax(sl, axis=-1)
    rows = i * BN + lax.broadcasted_iota(jnp.int32, (BN, 1), 0)
    sm = jnp.where(rows < N, sm, 0.0)
    spad_ref[...] = jnp.concatenate(
        [sm, jnp.zeros((BN, 15 - K), jnp.float32), jnp.ones((BN, 1), jnp.float32)],
        axis=1)
    logits = jnp.dot(h2, wc_ref[...], preferred_element_type=jnp.float32) + bc_ref[...]
    y_ref[...] = jax.nn.log_softmax(logits, axis=-1)

    @pl.when(i == 0)
    def _init():
        ss_ref[...] = jnp.zeros_like(ss_ref)

    ss_ref[...] += lax.dot_general(sm, sm, (((0,), (0,)), ((), ())),
                                   preferred_element_type=jnp.float32)


def _t4_body(spad_ref, as_ref, ss_ref, mc_ref, ol_ref, num_acc, den_acc):
    i = pl.program_id(0)

    @pl.when(i == 0)
    def _init():
        num_acc[0, 0] = 0.0
        den_acc[0, 0] = 0.0

    sv = spad_ref[...][:, :K]
    av = as_ref[0] + as_ref[1]
    num_acc[0, 0] += jnp.sum(sv * av[:, :K])
    den_acc[0, 0] += jnp.sum(av[:, 15:16] * jnp.sum(sv * sv, axis=1, keepdims=True))

    @pl.when(i == pl.num_programs(0) - 1)
    def _fin():
        mc_ref[0, 0] = -(num_acc[0, 0] / den_acc[0, 0])
        ssv = ss_ref[...]
        nss = jnp.sqrt(jnp.sum(ssv * ssv))
        r = lax.broadcasted_iota(jnp.int32, (K, K), 0)
        cc = lax.broadcasted_iota(jnp.int32, (K, K), 1)
        eye = jnp.where(r == cc, 1.0 / jnp.sqrt(jnp.float32(K)), 0.0)
        t = ssv / nss - eye
        ol_ref[0, 0] = jnp.sqrt(jnp.sum(t * t))


_t1 = pl.pallas_call(
    _t1_body,
    grid=(GRID,),
    in_specs=[
        pl.BlockSpec((NC, BN, 16), lambda i: (0, i, 0)),
        pl.BlockSpec((BN, D_IN), lambda i: (i, 0)),
        pl.BlockSpec((D_IN, H), lambda i: (0, 0)),
    ],
    out_specs=[
        pl.BlockSpec((BN, H), lambda i: (i, 0)),
        pl.BlockSpec((BN, 16), lambda i: (i, 0)),
    ],
    out_shape=[
        jax.ShapeDtypeStruct((NP, H), jnp.float32),
        jax.ShapeDtypeStruct((NP, 16), jnp.float32),
    ],
)

_t2 = pl.pallas_call(
    _t2_body,
    grid=(GRID,),
    in_specs=[
        pl.BlockSpec((NC, BN, H), lambda i: (0, i, 0)),
        pl.BlockSpec((BN, H), lambda i: (i, 0)),
        pl.BlockSpec((BN, 16), lambda i: (i, 0)),
        pl.BlockSpec((1, H), lambda i: (0, 0)),
        pl.BlockSpec((H, H), lambda i: (0, 0)),
    ],
    out_specs=pl.BlockSpec((BN, H), lambda i: (i, 0)),
    out_shape=jax.ShapeDtypeStruct((NP, H), jnp.float32),
)

_t3 = pl.pallas_call(
    _t3_body,
    grid=(GRID,),
    in_specs=[
        pl.BlockSpec((NC, BN, H), lambda i: (0, i, 0)),
        pl.BlockSpec((BN, H), lambda i: (i, 0)),
        pl.BlockSpec((BN, 16), lambda i: (i, 0)),
        pl.BlockSpec((1, H), lambda i: (0, 0)),
        pl.BlockSpec((H, K), lambda i: (0, 0)),
        pl.BlockSpec((1, K), lambda i: (0, 0)),
        pl.BlockSpec((H, C_OUT), lambda i: (0, 0)),
        pl.BlockSpec((1, C_OUT), lambda i: (0, 0)),
    ],
    out_specs=[
        pl.BlockSpec((BN, C_OUT), lambda i: (i, 0)),
        pl.BlockSpec((BN, 16), lambda i: (i, 0)),
        pl.BlockSpec((K, K), lambda i: (0, 0)),
    ],
    out_shape=[
        jax.ShapeDtypeStruct((NP, C_OUT), jnp.float32),
        jax.ShapeDtypeStruct((NP, 16), jnp.float32),
        jax.ShapeDtypeStruct((K, K), jnp.float32),
    ],
)

_t4 = pl.pallas_call(
    _t4_body,
    grid=(GRID,),
    in_specs=[
        pl.BlockSpec((BN, 16), lambda i: (i, 0)),
        pl.BlockSpec((NC, BN, 16), lambda i: (0, i, 0)),
        pl.BlockSpec((K, K), lambda i: (0, 0)),
    ],
    out_specs=[
        pl.BlockSpec(memory_space=pltpu.SMEM),
        pl.BlockSpec(memory_space=pltpu.SMEM),
    ],
    out_shape=[
        jax.ShapeDtypeStruct((1, 1), jnp.float32),
        jax.ShapeDtypeStruct((1, 1), jnp.float32),
    ],
    scratch_shapes=[pltpu.SMEM((1, 1), jnp.float32),
                    pltpu.SMEM((1, 1), jnp.float32)],
)


def kernel(x, edge_index, W1, b1, W2, b2, Wp, bp, Wc, bc):
    src = edge_index[0]
    dst = edge_index[1]
    pad = jnp.full((EP - E,), N, jnp.int32)
    srcp = jnp.concatenate([src, pad]).reshape(TOTCH, CH)
    dstp = jnp.concatenate([dst, pad]).reshape(TOTCH, CH)
    xp = jnp.pad(x, ((0, NP - N), (0, 0)))
    zeros16 = jnp.zeros((NP, 16), jnp.float32)
    zeros32 = jnp.zeros((NP, H), jnp.float32)
    ones = jnp.ones((CH, 16), jnp.float32)

    degin = _sc_degrees_kernel()(dstp, ones, zeros16)
    g1, dis = _t1(degin, xp, W1)
    seg1 = _segsum_kernel(H)(g1, srcp, dstp, zeros32)
    g2 = _t2(seg1, g1, dis, b1.reshape(1, H), W2)
    seg2 = _segsum_kernel(H)(g2, srcp, dstp, zeros32)
    y, spad, ss = _t3(seg2, g2, dis, b2.reshape(1, H),
                      Wp, bp.reshape(1, K), Wc, bc.reshape(1, C_OUT))
    a_s = _segsum_kernel(16)(spad, dstp, srcp, zeros16)
    mc, ol = _t4(spad, a_s, ss)
    return (y[:N], mc[0, 0], ol[0, 0])


# speedup vs baseline: 1.0902x; 1.0014x over previous
import functools

import jax
import jax.numpy as jnp
from jax import lax
from jax.experimental import pallas as pl
from jax.experimental.pallas import tpu as pltpu
from jax.experimental.pallas import tpu_sc as plsc

N = 10000
D_IN = 128
H = 32
K = 10
C_OUT = 7
E = 160000

NC = 2
NS = 16
NW = NC * NS
CH = 128
EWP = 5120
EP = EWP * NW
NCHUNK = EWP // CH
KG = 8
NG = NCHUNK // KG
TOTCH = EP // CH
NP = 10240
RPS = NP // NS
BN = 2048
GRID = NP // BN


@functools.cache
def _sc_mesh():
    return plsc.VectorSubcoreMesh(core_axis_name="c", subcore_axis_name="s",
                                  num_cores=NC, num_subcores=NS)


@functools.cache
def _sc_degrees_kernel():
    @functools.partial(
        pl.kernel,
        out_type=jax.ShapeDtypeStruct((NC, NP, 16), jnp.float32),
        mesh=_sc_mesh(),
        compiler_params=pltpu.CompilerParams(use_tc_tiling_on_sc=False),
        scratch_types=[
            pltpu.VMEM((NCHUNK, CH), jnp.int32),
            pltpu.VMEM((CH, 16), jnp.float32),
            pltpu.VMEM_SHARED((NP, 16), jnp.float32),
            pltpu.SemaphoreType.DMA,
        ],
    )
    def _sc_degrees(dst_h, ones_h, zeros_h, degin_h,
                    di_v, ones_v, accin_s, sem_in):
        c = lax.axis_index("c")
        s = lax.axis_index("s")
        wid = s * NC + c
        pltpu.sync_copy(zeros_h.at[pl.ds(s * RPS, RPS)], accin_s.at[pl.ds(s * RPS, RPS)])
        pltpu.sync_copy(ones_h, ones_v)
        pltpu.sync_copy(dst_h.at[pl.ds(wid * NCHUNK, NCHUNK)], di_v)
        plsc.subcore_barrier()

        for g in range(NG):
            ds = []
            for j in range(KG):
                t = g * KG + j
                ds.append(pltpu.async_copy(ones_v, accin_s.at[di_v.at[t]],
                                           sem_in, add=True))
            for d in ds:
                d.wait()
        plsc.subcore_barrier()
        pltpu.sync_copy(accin_s.at[pl.ds(s * RPS, RPS)], degin_h.at[c, pl.ds(s * RPS, RPS)])

    return _sc_degrees


@functools.cache
def _segsum_kernel(width):
    @functools.partial(
        pl.kernel,
        out_type=jax.ShapeDtypeStruct((NC, NP, width), jnp.float32),
        mesh=_sc_mesh(),
        compiler_params=pltpu.CompilerParams(use_tc_tiling_on_sc=False),
        scratch_types=[
            pltpu.VMEM((NCHUNK, CH), jnp.int32),
            pltpu.VMEM((NCHUNK, CH), jnp.int32),
            pltpu.VMEM((KG, CH, width), jnp.float32),
            pltpu.VMEM((KG, CH, width), jnp.float32),
            pltpu.VMEM_SHARED((NP, width), jnp.float32),
            pltpu.SemaphoreType.DMA,
            pltpu.SemaphoreType.DMA,
            pltpu.SemaphoreType.DMA,
            pltpu.SemaphoreType.DMA,
        ],
    )
    def _segsum(tab_h, gidx_h, sidx_h, zeros_h, out_h,
                gi_v, si_v, rows_a, rows_b, acc_s, gsa, gsb, ssa, ssb):
        c = lax.axis_index("c")
        s = lax.axis_index("s")
        wid = s * NC + c
        base = wid * NCHUNK
        pltpu.sync_copy(zeros_h.at[pl.ds(s * RPS, RPS)], acc_s.at[pl.ds(s * RPS, RPS)])
        pltpu.sync_copy(gidx_h.at[pl.ds(base, NCHUNK)], gi_v)
        pltpu.sync_copy(sidx_h.at[pl.ds(base, NCHUNK)], si_v)
        plsc.subcore_barrier()

        rows = (rows_a, rows_b)
        gsem = (gsa, gsb)
        ssem = (ssa, ssb)
        gdesc = [[], []]
        sdesc = [[], []]

        def fire_gathers(g):
            b = g % 2
            gdesc[b] = [
                pltpu.async_copy(tab_h.at[gi_v.at[g * KG + j]], rows[b].at[j], gsem[b])
                for j in range(KG)
            ]

        fire_gathers(0)
        for g in range(NG):
            b = g % 2
            if g + 1 < NG:
                for d in sdesc[1 - b]:
                    d.wait()
                fire_gathers(g + 1)
            for d in gdesc[b]:
                d.wait()
            sdesc[b] = [
                pltpu.async_copy(rows[b].at[j], acc_s.at[si_v.at[g * KG + j]],
                                 ssem[b], add=True)
                for j in range(KG)
            ]
        for b in (0, 1):
            for d in sdesc[b]:
                d.wait()
        plsc.subcore_barrier()
        pltpu.sync_copy(acc_s.at[pl.ds(s * RPS, RPS)], out_h.at[c, pl.ds(s * RPS, RPS)])

    return _segsum


def _t1_body(degin_ref, x_ref, w1_ref, g1_ref, dis_ref):
    deg = degin_ref[0] + degin_ref[1] + 1.0
    dis = lax.rsqrt(deg)
    dis_ref[...] = dis
    h = jnp.dot(x_ref[...], w1_ref[...], preferred_element_type=jnp.float32)
    g1_ref[...] = dis[:, 0:1] * h


def _t2_body(seg_ref, g1_ref, dis_ref, b1_ref, w2_ref, g2_ref):
    dis = dis_ref[...][:, 0:1]
    h1 = jnp.maximum(dis * (seg_ref[0] + seg_ref[1] + g1_ref[...]) + b1_ref[...], 0.0)
    g2_ref[...] = dis * jnp.dot(h1, w2_ref[...], preferred_element_type=jnp.float32)


def _t3_body(seg_ref, g2_ref, dis_ref, b2_ref, wp_ref, bp_ref,
             wc_ref, bc_ref, y_ref, spad_ref, ss_ref):
    i = pl.program_id(0)
    dis = dis_ref[...][:, 0:1]
    h2 = jnp.maximum(dis * (seg_ref[0] + seg_ref[1] + g2_ref[...]) + b2_ref[...], 0.0)
    sl = jnp.dot(h2, wp_ref[...], preferred_element_type=jnp.float32) + bp_ref[...]
    sm = jax.nn.softmax(sl, axis=-1)
    rows = i * BN + lax.broadcasted_iota(jnp.int32, (BN, 1), 0)
    sm = jnp.where(rows < N, sm, 0.0)
    spad_ref[...] = jnp.concatenate(
        [sm, jnp.zeros((BN, 15 - K), jnp.float32), jnp.ones((BN, 1), jnp.float32)],
        axis=1)
    logits = jnp.dot(h2, wc_ref[...], preferred_element_type=jnp.float32) + bc_ref[...]
    y_ref[...] = jax.nn.log_softmax(logits, axis=-1)

    @pl.when(i == 0)
    def _init():
        ss_ref[...] = jnp.zeros_like(ss_ref)

    ss_ref[...] += lax.dot_general(sm, sm, (((0,), (0,)), ((), ())),
                                   preferred_element_type=jnp.float32)


def _t4_body(spad_ref, as_ref, ss_ref, mc_ref, ol_ref, num_acc, den_acc):
    i = pl.program_id(0)

    @pl.when(i == 0)
    def _init():
        num_acc[0, 0] = 0.0
        den_acc[0, 0] = 0.0

    sv = spad_ref[...][:, :K]
    av = as_ref[0] + as_ref[1]
    num_acc[0, 0] += jnp.sum(sv * av[:, :K])
    den_acc[0, 0] += jnp.sum(av[:, 15:16] * jnp.sum(sv * sv, axis=1, keepdims=True))

    @pl.when(i == pl.num_programs(0) - 1)
    def _fin():
        mc_ref[0, 0] = -(num_acc[0, 0] / den_acc[0, 0])
        ssv = ss_ref[...]
        nss = jnp.sqrt(jnp.sum(ssv * ssv))
        r = lax.broadcasted_iota(jnp.int32, (K, K), 0)
        cc = lax.broadcasted_iota(jnp.int32, (K, K), 1)
        eye = jnp.where(r == cc, 1.0 / jnp.sqrt(jnp.float32(K)), 0.0)
        t = ssv / nss - eye
        ol_ref[0, 0] = jnp.sqrt(jnp.sum(t * t))


_t1 = pl.pallas_call(
    _t1_body,
    grid=(GRID,),
    in_specs=[
        pl.BlockSpec((NC, BN, 16), lambda i: (0, i, 0)),
        pl.BlockSpec((BN, D_IN), lambda i: (i, 0)),
        pl.BlockSpec((D_IN, H), lambda i: (0, 0)),
    ],
    out_specs=[
        pl.BlockSpec((BN, H), lambda i: (i, 0)),
        pl.BlockSpec((BN, 16), lambda i: (i, 0)),
    ],
    out_shape=[
        jax.ShapeDtypeStruct((NP, H), jnp.float32),
        jax.ShapeDtypeStruct((NP, 16), jnp.float32),
    ],
)

_t2 = pl.pallas_call(
    _t2_body,
    grid=(GRID,),
    in_specs=[
        pl.BlockSpec((NC, BN, H), lambda i: (0, i, 0)),
        pl.BlockSpec((BN, H), lambda i: (i, 0)),
        pl.BlockSpec((BN, 16), lambda i: (i, 0)),
        pl.BlockSpec((1, H), lambda i: (0, 0)),
        pl.BlockSpec((H, H), lambda i: (0, 0)),
    ],
    out_specs=pl.BlockSpec((BN, H), lambda i: (i, 0)),
    out_shape=jax.ShapeDtypeStruct((NP, H), jnp.float32),
)

_t3 = pl.pallas_call(
    _t3_body,
    grid=(GRID,),
    in_specs=[
        pl.BlockSpec((NC, BN, H), lambda i: (0, i, 0)),
        pl.BlockSpec((BN, H), lambda i: (i, 0)),
        pl.BlockSpec((BN, 16), lambda i: (i, 0)),
        pl.BlockSpec((1, H), lambda i: (0, 0)),
        pl.BlockSpec((H, K), lambda i: (0, 0)),
        pl.BlockSpec((1, K), lambda i: (0, 0)),
        pl.BlockSpec((H, C_OUT), lambda i: (0, 0)),
        pl.BlockSpec((1, C_OUT), lambda i: (0, 0)),
    ],
    out_specs=[
        pl.BlockSpec((BN, C_OUT), lambda i: (i, 0)),
        pl.BlockSpec((BN, 16), lambda i: (i, 0)),
        pl.BlockSpec((K, K), lambda i: (0, 0)),
    ],
    out_shape=[
        jax.ShapeDtypeStruct((NP, C_OUT), jnp.float32),
        jax.ShapeDtypeStruct((NP, 16), jnp.float32),
        jax.ShapeDtypeStruct((K, K), jnp.float32),
    ],
)

_t4 = pl.pallas_call(
    _t4_body,
    grid=(GRID,),
    in_specs=[
        pl.BlockSpec((BN, 16), lambda i: (i, 0)),
        pl.BlockSpec((NC, BN, 16), lambda i: (0, i, 0)),
        pl.BlockSpec((K, K), lambda i: (0, 0)),
    ],
    out_specs=[
        pl.BlockSpec(memory_space=pltpu.SMEM),
        pl.BlockSpec(memory_space=pltpu.SMEM),
    ],
    out_shape=[
        jax.ShapeDtypeStruct((1, 1), jnp.float32),
        jax.ShapeDtypeStruct((1, 1), jnp.float32),
    ],
    scratch_shapes=[pltpu.SMEM((1, 1), jnp.float32),
                    pltpu.SMEM((1, 1), jnp.float32)],
)


def kernel(x, edge_index, W1, b1, W2, b2, Wp, bp, Wc, bc):
    src = edge_index[0]
    dst = edge_index[1]
    pad = jnp.full((EP - E,), N, jnp.int32)
    srcp = jnp.concatenate([src, pad]).reshape(TOTCH, CH)
    dstp = jnp.concatenate([dst, pad]).reshape(TOTCH, CH)
    xp = jnp.pad(x, ((0, NP - N), (0, 0)))
    zeros16 = jnp.zeros((NP, 16), jnp.float32)
    zeros32 = jnp.zeros((NP, H), jnp.float32)
    ones = jnp.ones((CH, 16), jnp.float32)

    degin = _sc_degrees_kernel()(dstp, ones, zeros16)
    g1, dis = _t1(degin, xp, W1)
    seg1 = _segsum_kernel(H)(g1, srcp, dstp, zeros32)
    g2 = _t2(seg1, g1, dis, b1.reshape(1, H), W2)
    seg2 = _segsum_kernel(H)(g2, srcp, dstp, zeros32)
    y, spad, ss = _t3(seg2, g2, dis, b2.reshape(1, H),
                      Wp, bp.reshape(1, K), Wc, bc.reshape(1, C_OUT))
    a_s = _segsum_kernel(16)(spad, dstp, srcp, zeros16)
    mc, ol = _t4(spad, a_s, ss)
    return (y[:N], mc[0, 0], ol[0, 0])
